# baseline (speedup 1.0000x reference)
import functools

import jax
import jax.numpy as jnp
from jax import lax
from jax.experimental import pallas as pl
from jax.experimental.pallas import tpu as pltpu
from jax.experimental.pallas import tpu_sc as plsc

_NC = 2
_NS = 16
_NW = _NC * _NS

_SUB = 2560


def _matmul_tc(x, w):
    def body(x_ref, w_ref, o_ref):
        o_ref[...] = jnp.dot(x_ref[...], w_ref[...],
                             preferred_element_type=jnp.float32)
    return pl.pallas_call(
        body,
        out_shape=jax.ShapeDtypeStruct((x.shape[0], w.shape[1]), jnp.float32),
    )(x, w)


def _add_relu_tc(p):
    def body(p_ref, o_ref):
        o_ref[...] = jnp.maximum(p_ref[0] + p_ref[1], 0.0)
    return pl.pallas_call(
        body,
        out_shape=jax.ShapeDtypeStruct(p.shape[1:], jnp.float32),
    )(p)


def _add_matmul_grouped_tc(q, w2b):
    def body(q_ref, w_ref, o_ref):
        s = q_ref[0] + q_ref[1]
        o_ref[...] = jnp.dot(s, w_ref[...], preferred_element_type=jnp.float32)
    return pl.pallas_call(
        body,
        out_shape=jax.ShapeDtypeStruct((q.shape[1], w2b.shape[1]), jnp.float32),
    )(q, w2b)


def _make_aggregate(NP, E2, D, idx_shift):
    epw = E2 // _NW
    B = 2560
    nblk = epw // B
    nsub = B // _SUB
    zrows = (NP // _NS) // 8 * 8
    tail = NP - zrows * _NS
    arows = -(-NP // _NS)
    assert epw * _NW == E2 and nblk * B == epw and nsub * _SUB == B
    assert 0 <= tail <= B and zrows <= B and tail % 8 == 0 and arows <= B

    mesh = plsc.VectorSubcoreMesh(core_axis_name="c", subcore_axis_name="s")

    @functools.partial(
        pl.kernel,
        out_type=jax.ShapeDtypeStruct((_NC, NP, D), jnp.float32),
        mesh=mesh,
        scratch_types=[
            pltpu.VMEM((2, nsub, _SUB), jnp.int32),
            pltpu.VMEM((2, nsub, _SUB), jnp.int32),
            pltpu.VMEM((2, B), jnp.float32),
            pltpu.VMEM((2, B, D), jnp.float32),
            pltpu.VMEM_SHARED((NP, D), jnp.float32),
            pltpu.SemaphoreType.DMA,
            pltpu.SemaphoreType.DMA,
            pltpu.SemaphoreType.DMA,
            pltpu.SemaphoreType.DMA,
        ],
        compiler_params=pltpu.CompilerParams(use_tc_tiling_on_sc=False),
    )
    def agg(table, edges3, w_hbm, out, colv, rowv, wv, rowsv, acc,
            gsem0, gsem1, ssem0, ssem1):
        cid = lax.axis_index("c")
        sid = lax.axis_index("s")
        wid = sid * _NC + cid
        gsems = (gsem0, gsem1)
        ssems = (ssem0, ssem1)

        def zbody(j, c):
            rowsv[0, j, :] = jnp.zeros((D,), jnp.float32)
            return c
        lax.fori_loop(0, arows, zbody, 0)
        pltpu.sync_copy(rowsv.at[0, pl.ds(0, arows)],
                        acc.at[pl.ds(sid * arows, arows)])
        plsc.subcore_barrier()

        def fire_gather(b):
            s = b % 2
            r0 = pl.multiple_of((wid * epw + b * B) // _SUB, nsub)
            off = pl.multiple_of(wid * epw + b * B, B)
            pltpu.sync_copy(edges3.at[1, pl.ds(r0, nsub)], colv.at[s])
            pltpu.sync_copy(edges3.at[0, pl.ds(r0, nsub)], rowv.at[s])
            pltpu.sync_copy(w_hbm.at[pl.ds(off, B)], wv.at[s])
            if idx_shift:
                def shbody(g, cc):
                    g16 = pl.multiple_of(g * 16, 16)
                    for r in range(nsub):
                        colv[s, r, pl.ds(g16, 16)] = (
                            colv[s, r, pl.ds(g16, 16)] << idx_shift)
                    return cc
                lax.fori_loop(0, _SUB // 16, shbody, 0)
            for j in range(nsub):
                pltpu.async_copy(table.at[colv.at[s].at[j]],
                                 rowsv.at[s, pl.ds(j * _SUB, _SUB)], gsems[s])

        def fire_scatter(b):
            s = b % 2
            for j in range(nsub):
                pltpu.async_copy(rowsv.at[s, pl.ds(j * _SUB, _SUB)],
                                 acc.at[rowv.at[s].at[j]], ssems[s], add=True)

        def drain(b, sems):
            s = b % 2
            pltpu.make_async_copy(out.at[cid].at[pl.ds(0, B)],
                                  rowsv.at[s], sems[s]).wait()

        def scale(b):
            s = b % 2
            def sbody(g, cc):
                w16 = wv[s, pl.ds(pl.multiple_of(g * 16, 16), 16)]
                for l in range(16):
                    j = g * 16 + l
                    rowsv[s, j, :] = rowsv[s, j, :] * w16[l]
                return cc
            lax.fori_loop(0, B // 16, sbody, 0)

        fire_gather(0)
        sd_pending = None
        for b in range(nblk):
            if b + 1 < nblk:
                if sd_pending is not None:
                    drain(sd_pending, ssems)
                    sd_pending = None
                fire_gather(b + 1)
            drain(b, gsems)
            scale(b)
            if sd_pending is not None:
                drain(sd_pending, ssems)
            fire_scatter(b)
            sd_pending = b
        drain(sd_pending, ssems)

        plsc.subcore_barrier()
        pltpu.sync_copy(acc.at[pl.ds(sid * zrows, zrows)],
                        out.at[cid].at[pl.ds(sid * zrows, zrows)])
        if tail:
            @pl.when(sid == 0)
            def _():
                pltpu.sync_copy(acc.at[pl.ds(_NS * zrows, tail)],
                                out.at[cid].at[pl.ds(_NS * zrows, tail)])

    return agg


def kernel(x, edge_index, edge_weight, W1, W2):
    N = x.shape[0]
    E = edge_index.shape[1]
    D = W1.shape[1]
    DO = W2.shape[1]
    NP = -(-N // 8 // _NS) * 8 * _NS
    E2 = -(-E // (_SUB * _NW * 8)) * (_SUB * _NW * 8)
    grp = NP * D // 128

    spread = (jnp.arange(E2 - E, dtype=jnp.int32) * 8) % N
    e3 = jnp.concatenate(
        [edge_index, jnp.stack([spread, spread])], axis=1,
    ).reshape(2, E2 // _SUB, _SUB)
    wp = jnp.pad(edge_weight, (0, E2 - E))
    w1p = jnp.pad(W1, ((0, 0), (0, 128 - D)))
    w2b = jnp.kron(jnp.eye(8, dtype=W2.dtype), W2)

    agg1 = _make_aggregate(NP, E2, D, idx_shift=3)
    agg2 = _make_aggregate(NP, E2, D, idx_shift=0)

    h1w = _matmul_tc(x, w1p)
    p = agg1(h1w.reshape(N * 8, D), e3, wp)
    h2g = _add_relu_tc(p.reshape(_NC, grp, 128))
    q = agg2(h2g.reshape(NP, D), e3, wp)
    og = _add_matmul_grouped_tc(q.reshape(_NC, grp, 128), w2b)
    return og[:N // 8].reshape(N, DO)


# speedup vs baseline: 2.1783x; 2.1783x over previous
import functools

import jax
import jax.numpy as jnp
from jax import lax
from jax.experimental import pallas as pl
from jax.experimental.pallas import tpu as pltpu
from jax.experimental.pallas import tpu_sc as plsc

_NC = 2
_NS = 16
_NW = _NC * _NS

_SUB = 512
_B = 2560


def _matmul_tc(x, w):
    def body(x_ref, w_ref, o_ref):
        o_ref[...] = jnp.dot(x_ref[...], w_ref[...],
                             preferred_element_type=jnp.float32)
    return pl.pallas_call(
        body,
        out_shape=jax.ShapeDtypeStruct((x.shape[0], w.shape[1]), jnp.float32),
    )(x, w)


def _add_relu_tc(p):
    def body(p_ref, o_ref):
        o_ref[...] = jnp.maximum(p_ref[0] + p_ref[1], 0.0)
    return pl.pallas_call(
        body,
        out_shape=jax.ShapeDtypeStruct(p.shape[1:], jnp.float32),
    )(p)


def _add_matmul_grouped_tc(q, w2b):
    def body(q_ref, w_ref, o_ref):
        s = q_ref[0] + q_ref[1]
        o_ref[...] = jnp.dot(s, w_ref[...], preferred_element_type=jnp.float32)
    return pl.pallas_call(
        body,
        out_shape=jax.ShapeDtypeStruct((q.shape[1], w2b.shape[1]), jnp.float32),
    )(q, w2b)


def _make_aggregate(NP, E2, D, idx_shift):
    epw = E2 // _NW
    B = _B
    nblk = epw // B
    nsub = B // _SUB
    zrows = (NP // _NS) // 8 * 8
    tail = NP - zrows * _NS
    arows = -(-NP // _NS)
    assert epw * _NW == E2 and nblk * B == epw and nsub * _SUB == B
    assert 0 <= tail <= B and zrows <= B and tail % 8 == 0 and arows <= B

    mesh = plsc.VectorSubcoreMesh(core_axis_name="c", subcore_axis_name="s")

    @functools.partial(
        pl.kernel,
        out_type=jax.ShapeDtypeStruct((_NC, NP, D), jnp.float32),
        mesh=mesh,
        scratch_types=[
            pltpu.VMEM((2, nsub, _SUB), jnp.int32),
            pltpu.VMEM((2, nsub, _SUB), jnp.int32),
            pltpu.VMEM((2, B), jnp.float32),
            pltpu.VMEM((2, B, D), jnp.float32),
            pltpu.VMEM_SHARED((NP, D), jnp.float32),
            pltpu.SemaphoreType.DMA,
            pltpu.SemaphoreType.DMA,
            pltpu.SemaphoreType.DMA,
            pltpu.SemaphoreType.DMA,
        ],
        compiler_params=pltpu.CompilerParams(use_tc_tiling_on_sc=False),
    )
    def agg(table, edges3, w_hbm, out, colv, rowv, wv, rowsv, acc,
            gsem0, gsem1, ssem0, ssem1):
        cid = lax.axis_index("c")
        sid = lax.axis_index("s")
        wid = sid * _NC + cid
        gsems = (gsem0, gsem1)
        ssems = (ssem0, ssem1)

        def zbody(j, c):
            rowsv[0, j, :] = jnp.zeros((D,), jnp.float32)
            return c
        lax.fori_loop(0, arows, zbody, 0)
        pltpu.sync_copy(rowsv.at[0, pl.ds(0, arows)],
                        acc.at[pl.ds(sid * arows, arows)])
        plsc.subcore_barrier()

        def fire_gather(b):
            s = b % 2
            r0 = pl.multiple_of((wid * epw + b * B) // _SUB, nsub)
            off = pl.multiple_of(wid * epw + b * B, B)
            pltpu.sync_copy(edges3.at[1, pl.ds(r0, nsub)], colv.at[s])
            pltpu.sync_copy(edges3.at[0, pl.ds(r0, nsub)], rowv.at[s])
            pltpu.sync_copy(w_hbm.at[pl.ds(off, B)], wv.at[s])
            if idx_shift:
                def shbody(g, cc):
                    g16 = pl.multiple_of(g * 16, 16)
                    for r in range(nsub):
                        colv[s, r, pl.ds(g16, 16)] = (
                            colv[s, r, pl.ds(g16, 16)] << idx_shift)
                    return cc
                lax.fori_loop(0, _SUB // 16, shbody, 0)
            for j in range(nsub):
                pltpu.async_copy(table.at[colv.at[s].at[j]],
                                 rowsv.at[s, pl.ds(j * _SUB, _SUB)], gsems[s])

        def fire_scatter(b):
            s = b % 2
            for j in range(nsub):
                pltpu.async_copy(rowsv.at[s, pl.ds(j * _SUB, _SUB)],
                                 acc.at[rowv.at[s].at[j]], ssems[s], add=True)

        def drain(b, sems):
            s = b % 2
            pltpu.make_async_copy(out.at[cid].at[pl.ds(0, B)],
                                  rowsv.at[s], sems[s]).wait()

        def scale(b):
            s = b % 2
            def sbody(g, cc):
                w16 = wv[s, pl.ds(pl.multiple_of(g * 16, 16), 16)]
                for l in range(16):
                    j = g * 16 + l
                    rowsv[s, j, :] = rowsv[s, j, :] * w16[l]
                return cc
            lax.fori_loop(0, B // 16, sbody, 0)

        fire_gather(0)
        sd_pending = None
        for b in range(nblk):
            if b + 1 < nblk:
                if sd_pending is not None:
                    drain(sd_pending, ssems)
                    sd_pending = None
                fire_gather(b + 1)
            drain(b, gsems)
            scale(b)
            if sd_pending is not None:
                drain(sd_pending, ssems)
            fire_scatter(b)
            sd_pending = b
        drain(sd_pending, ssems)

        plsc.subcore_barrier()
        pltpu.sync_copy(acc.at[pl.ds(sid * zrows, zrows)],
                        out.at[cid].at[pl.ds(sid * zrows, zrows)])
        if tail:
            @pl.when(sid == 0)
            def _():
                pltpu.sync_copy(acc.at[pl.ds(_NS * zrows, tail)],
                                out.at[cid].at[pl.ds(_NS * zrows, tail)])

    return agg


def kernel(x, edge_index, edge_weight, W1, W2):
    N = x.shape[0]
    E = edge_index.shape[1]
    D = W1.shape[1]
    DO = W2.shape[1]
    NP = -(-N // 8 // _NS) * 8 * _NS
    E2 = -(-E // (_B * _NW)) * (_B * _NW)
    grp = NP * D // 128

    spread = (jnp.arange(E2 - E, dtype=jnp.int32) * 8) % N
    e3 = jnp.concatenate(
        [edge_index, jnp.stack([spread, spread])], axis=1,
    ).reshape(2, E2 // _SUB, _SUB)
    wp = jnp.pad(edge_weight, (0, E2 - E))
    w1p = jnp.pad(W1, ((0, 0), (0, 128 - D)))
    w2b = jnp.kron(jnp.eye(8, dtype=W2.dtype), W2)

    agg1 = _make_aggregate(NP, E2, D, idx_shift=3)
    agg2 = _make_aggregate(NP, E2, D, idx_shift=0)

    h1w = _matmul_tc(x, w1p)
    p = agg1(h1w.reshape(N * 8, D), e3, wp)
    h2g = _add_relu_tc(p.reshape(_NC, grp, 128))
    q = agg2(h2g.reshape(NP, D), e3, wp)
    og = _add_matmul_grouped_tc(q.reshape(_NC, grp, 128), w2b)
    return og[:N // 8].reshape(N, DO)
